# baseline (speedup 1.0000x reference)
import functools

import jax
import jax.numpy as jnp
from jax import lax
from jax.experimental import pallas as pl
from jax.experimental.pallas import tpu as pltpu
from jax.experimental.pallas import tpu_sc as plsc

EMBED = 768
HIDDEN = 3072
E = 8
T = 2048
BLK = 256
PP = T * 2 + E * BLK
NB = PP // BLK
HB = 768
NH = HIDDEN // HB

NC, NS = 2, 16
NW = NC * NS
SLOTS_W = PP // NW
SCHUNK = SLOTS_W // 2
TOK_W = T // NW


def _router_body(x_ref, wg_ref, g_ref):
    logits = lax.dot_general(x_ref[...], wg_ref[...], (((1,), (1,)), ((), ())),
                             preferred_element_type=jnp.float32)
    m = jnp.max(logits, axis=1, keepdims=True)
    p = jnp.exp(logits - m)
    p = p / jnp.sum(p, axis=1, keepdims=True)
    iot = lax.broadcasted_iota(jnp.int32, (T, E), 1)
    m1 = jnp.max(p, axis=1, keepdims=True)
    i1 = jnp.min(jnp.where(p == m1, iot, E), axis=1, keepdims=True)
    sel1 = iot == i1
    p2 = jnp.where(sel1, -1.0, p)
    m2 = jnp.max(p2, axis=1, keepdims=True)
    i2 = jnp.min(jnp.where(p2 == m2, iot, E), axis=1, keepdims=True)
    sel2 = iot == i2
    den = m1 + m2 + 1e-9
    g_ref[...] = (jnp.where(sel1, m1, 0.0) + jnp.where(sel2, m2, 0.0)) / den


def _router(xf, Wg):
    return pl.pallas_call(
        _router_body,
        out_shape=jax.ShapeDtypeStruct((T, E), jnp.float32),
    )(xf, Wg)


def _dispatch_body(x_hbm, src_hbm, xs_hbm, idx_v, rows_v, sem):
    wid = lax.axis_index("s") * NC + lax.axis_index("c")
    for cc in range(SLOTS_W // SCHUNK):
        base = wid * SLOTS_W + cc * SCHUNK
        pltpu.sync_copy(src_hbm.at[pl.ds(base, SCHUNK)], idx_v)
        pltpu.async_copy(x_hbm.at[idx_v], rows_v, sem).wait()
        pltpu.sync_copy(rows_v, xs_hbm.at[pl.ds(base, SCHUNK)])


def _dispatch(xf, src_tok):
    mesh = plsc.VectorSubcoreMesh(core_axis_name="c", subcore_axis_name="s")
    fn = functools.partial(
        pl.kernel, mesh=mesh,
        out_type=jax.ShapeDtypeStruct((PP, EMBED), jnp.float32),
        scratch_types=[
            pltpu.VMEM((SCHUNK,), jnp.int32),
            pltpu.VMEM((SCHUNK, EMBED), jnp.float32),
            pltpu.SemaphoreType.DMA,
        ],
    )(_dispatch_body)
    return fn(xf, src_tok)


def _ffn_body(be_ref, xs_ref, gs_ref, w1_ref, b1_ref, w2_ref, b2_ref, out_ref):
    j = pl.program_id(1)
    h = lax.dot_general(xs_ref[...], w1_ref[0], (((1,), (1,)), ((), ())),
                        preferred_element_type=jnp.float32)
    h = jnp.maximum(h + b1_ref[0, 0][None, :], 0.0)
    yp = lax.dot_general(h, w2_ref[0], (((1,), (1,)), ((), ())),
                         preferred_element_type=jnp.float32)

    @pl.when(j == 0)
    def _():
        out_ref[...] = yp

    @pl.when(j > 0)
    def _():
        out_ref[...] = out_ref[...] + yp

    @pl.when(j == NH - 1)
    def _():
        g = gs_ref[0, 0, :].reshape(BLK, 1)
        out_ref[...] = (out_ref[...] + b2_ref[0, 0][None, :]) * g


def _ffn(be, xs, gs3, W1, b1r, W2, b2r):
    grid_spec = pltpu.PrefetchScalarGridSpec(
        num_scalar_prefetch=1,
        grid=(NB, NH),
        in_specs=[
            pl.BlockSpec((BLK, EMBED), lambda i, j, be: (i, 0)),
            pl.BlockSpec((1, 1, BLK), lambda i, j, be: (i, 0, 0)),
            pl.BlockSpec((1, HB, EMBED), lambda i, j, be: (be[i], j, 0)),
            pl.BlockSpec((1, 1, HB), lambda i, j, be: (be[i], 0, j)),
            pl.BlockSpec((1, EMBED, HB), lambda i, j, be: (be[i], 0, j)),
            pl.BlockSpec((1, 1, EMBED), lambda i, j, be: (be[i], 0, 0)),
        ],
        out_specs=pl.BlockSpec((BLK, EMBED), lambda i, j, be: (i, 0)),
    )
    return pl.pallas_call(
        _ffn_body,
        grid_spec=grid_spec,
        out_shape=jax.ShapeDtypeStruct((PP, EMBED), jnp.float32),
        compiler_params=pltpu.CompilerParams(
            dimension_semantics=("parallel", "arbitrary")),
    )(be, xs, gs3, W1, b1r, W2, b2r)


def _combine_body(ys_hbm, dst0_hbm, dst1_hbm, out_hbm, i0_v, i1_v, a_v, b_v, sem):
    wid = lax.axis_index("s") * NC + lax.axis_index("c")
    base = wid * TOK_W
    pltpu.sync_copy(dst0_hbm.at[pl.ds(base, TOK_W)], i0_v)
    pltpu.sync_copy(dst1_hbm.at[pl.ds(base, TOK_W)], i1_v)
    pltpu.async_copy(ys_hbm.at[i0_v], a_v, sem).wait()
    pltpu.async_copy(ys_hbm.at[i1_v], b_v, sem).wait()

    def body(t, carry):
        for c in range(EMBED // 16):
            sl = pl.ds(c * 16, 16)
            a_v[t, sl] = a_v[t, sl] + b_v[t, sl]
        return carry

    lax.fori_loop(0, TOK_W, body, 0)
    pltpu.sync_copy(a_v, out_hbm.at[pl.ds(base, TOK_W)])


def _combine(ys, dst0, dst1):
    mesh = plsc.VectorSubcoreMesh(core_axis_name="c", subcore_axis_name="s")
    fn = functools.partial(
        pl.kernel, mesh=mesh,
        out_type=jax.ShapeDtypeStruct((T, EMBED), jnp.float32),
        scratch_types=[
            pltpu.VMEM((TOK_W,), jnp.int32),
            pltpu.VMEM((TOK_W,), jnp.int32),
            pltpu.VMEM((TOK_W, EMBED), jnp.float32),
            pltpu.VMEM((TOK_W, EMBED), jnp.float32),
            pltpu.SemaphoreType.DMA,
        ],
    )(_combine_body)
    return fn(ys, dst0, dst1)


def _bookkeeping(gates):
    topw, topi = lax.top_k(gates, 2)
    e0, e1 = topi[:, 0], topi[:, 1]
    w0, w1 = topw[:, 0], topw[:, 1]
    oh0 = jax.nn.one_hot(e0, E, dtype=jnp.int32)
    oh1 = jax.nn.one_hot(e1, E, dtype=jnp.int32)
    oh = oh0 + oh1
    cum = jnp.cumsum(oh, axis=0) - oh
    counts = jnp.sum(oh, axis=0)
    nblk_e = (counts + BLK - 1) // BLK
    blk_start = jnp.concatenate(
        [jnp.zeros((1,), jnp.int32), jnp.cumsum(nblk_e)[:-1].astype(jnp.int32)])
    off = blk_start * BLK
    rank0 = jnp.take_along_axis(cum, e0[:, None], axis=1)[:, 0]
    rank1 = jnp.take_along_axis(cum, e1[:, None], axis=1)[:, 0]
    dst0 = (jnp.take(off, e0) + rank0).astype(jnp.int32)
    dst1 = (jnp.take(off, e1) + rank1).astype(jnp.int32)
    tok = jnp.arange(T, dtype=jnp.int32)
    src_tok = jnp.zeros((PP,), jnp.int32).at[dst0].set(tok).at[dst1].set(tok)
    g_slot = jnp.zeros((PP,), jnp.float32).at[dst0].set(w0).at[dst1].set(w1)
    bidx = jnp.arange(NB, dtype=jnp.int32)
    be = jnp.clip(jnp.sum((bidx[:, None] >= blk_start[None, :]).astype(jnp.int32),
                          axis=1) - 1, 0, E - 1).astype(jnp.int32)
    gs3 = g_slot.reshape(NB, 1, BLK)
    return src_tok, gs3, be, dst0, dst1


def kernel(x, Wg, W1, b1, W2, b2):
    orig_shape = x.shape
    xf = x.reshape(T, EMBED)
    gates = _router(xf, Wg)
    src_tok, gs3, be, dst0, dst1 = _bookkeeping(gates)
    xs = _dispatch(xf, src_tok)
    b1r = b1.reshape(E, 1, HIDDEN)
    b2r = b2.reshape(E, 1, EMBED)
    ys = _ffn(be, xs, gs3, W1, b1r, W2, b2r)
    out = _combine(ys, dst0, dst1)
    return out.reshape(orig_shape)


# speedup vs baseline: 1.1586x; 1.1586x over previous
import functools

import jax
import jax.numpy as jnp
from jax import lax
from jax.experimental import pallas as pl
from jax.experimental.pallas import tpu as pltpu
from jax.experimental.pallas import tpu_sc as plsc

EMBED = 768
HIDDEN = 3072
E = 8
T = 2048
BLK = 128
PP = T * 2 + E * BLK
NB = PP // BLK

NC, NS = 2, 16
NW = NC * NS
SLOTS_W = PP // NW
SCHUNK = SLOTS_W // 2
TOK_W = T // NW


def _router_body(x_ref, wg_ref, g_ref):
    logits = lax.dot_general(x_ref[...], wg_ref[...], (((1,), (1,)), ((), ())),
                             preferred_element_type=jnp.float32)
    m = jnp.max(logits, axis=1, keepdims=True)
    p = jnp.exp(logits - m)
    p = p / jnp.sum(p, axis=1, keepdims=True)
    iot = lax.broadcasted_iota(jnp.int32, (T, E), 1)
    m1 = jnp.max(p, axis=1, keepdims=True)
    i1 = jnp.min(jnp.where(p == m1, iot, E), axis=1, keepdims=True)
    sel1 = iot == i1
    p2 = jnp.where(sel1, -1.0, p)
    m2 = jnp.max(p2, axis=1, keepdims=True)
    i2 = jnp.min(jnp.where(p2 == m2, iot, E), axis=1, keepdims=True)
    sel2 = iot == i2
    den = m1 + m2 + 1e-9
    g_ref[...] = (jnp.where(sel1, m1, 0.0) + jnp.where(sel2, m2, 0.0)) / den


def _router(xf, Wg):
    return pl.pallas_call(
        _router_body,
        out_shape=jax.ShapeDtypeStruct((T, E), jnp.float32),
    )(xf, Wg)


def _dispatch_body(x_hbm, src_hbm, xs_hbm, idx_v, r0, r1, gsem, ssem):
    wid = lax.axis_index("s") * NC + lax.axis_index("c")
    wbase = wid * SLOTS_W
    pltpu.sync_copy(src_hbm.at[pl.ds(wbase, SLOTS_W)], idx_v)
    g0 = pltpu.async_copy(x_hbm.at[idx_v.at[pl.ds(0, SCHUNK)]], r0, gsem)
    g1 = pltpu.async_copy(x_hbm.at[idx_v.at[pl.ds(SCHUNK, SCHUNK)]], r1, gsem)
    g0.wait()
    s0 = pltpu.async_copy(r0, xs_hbm.at[pl.ds(wbase, SCHUNK)], ssem)
    g1.wait()
    s1 = pltpu.async_copy(r1, xs_hbm.at[pl.ds(wbase + SCHUNK, SCHUNK)], ssem)
    s0.wait()
    s1.wait()


def _dispatch(xf, src_tok):
    mesh = plsc.VectorSubcoreMesh(core_axis_name="c", subcore_axis_name="s")
    fn = functools.partial(
        pl.kernel, mesh=mesh,
        out_type=jax.ShapeDtypeStruct((PP, EMBED), jnp.float32),
        scratch_types=[
            pltpu.VMEM((SLOTS_W,), jnp.int32),
            pltpu.VMEM((SCHUNK, EMBED), jnp.float32),
            pltpu.VMEM((SCHUNK, EMBED), jnp.float32),
            pltpu.SemaphoreType.DMA,
            pltpu.SemaphoreType.DMA,
        ],
    )(_dispatch_body)
    return fn(xf, src_tok)


def _ffn_body(be_ref, xs_ref, gs_ref, w1_ref, b1_ref, w2_ref, b2_ref, out_ref,
              w1b_s, w2b_s):
    i = pl.program_id(0)
    prev = be_ref[jnp.maximum(i - 1, 0)]
    fresh = jnp.logical_or(i == 0, be_ref[i] != prev)

    @pl.when(fresh)
    def _():
        w1b_s[...] = w1_ref[0].astype(jnp.bfloat16)
        w2b_s[...] = w2_ref[0].astype(jnp.bfloat16)

    xb = xs_ref[...].astype(jnp.bfloat16)
    h = lax.dot_general(xb, w1b_s[...], (((1,), (1,)), ((), ())),
                        preferred_element_type=jnp.float32)
    h = jnp.maximum(h + b1_ref[0, 0][None, :], 0.0).astype(jnp.bfloat16)
    y = lax.dot_general(h, w2b_s[...], (((1,), (1,)), ((), ())),
                        preferred_element_type=jnp.float32)
    g = gs_ref[0, 0, :].reshape(BLK, 1)
    out_ref[...] = (y + b2_ref[0, 0][None, :]) * g


def _ffn(be, xs, gs3, W1, b1r, W2, b2r):
    grid_spec = pltpu.PrefetchScalarGridSpec(
        num_scalar_prefetch=1,
        grid=(NB,),
        in_specs=[
            pl.BlockSpec((BLK, EMBED), lambda i, be: (i, 0)),
            pl.BlockSpec((1, 1, BLK), lambda i, be: (i, 0, 0)),
            pl.BlockSpec((1, HIDDEN, EMBED), lambda i, be: (be[i], 0, 0)),
            pl.BlockSpec((1, 1, HIDDEN), lambda i, be: (be[i], 0, 0)),
            pl.BlockSpec((1, EMBED, HIDDEN), lambda i, be: (be[i], 0, 0)),
            pl.BlockSpec((1, 1, EMBED), lambda i, be: (be[i], 0, 0)),
        ],
        out_specs=pl.BlockSpec((BLK, EMBED), lambda i, be: (i, 0)),
        scratch_shapes=[
            pltpu.VMEM((HIDDEN, EMBED), jnp.bfloat16),
            pltpu.VMEM((EMBED, HIDDEN), jnp.bfloat16),
        ],
    )
    return pl.pallas_call(
        _ffn_body,
        grid_spec=grid_spec,
        out_shape=jax.ShapeDtypeStruct((PP, EMBED), jnp.float32),
        compiler_params=pltpu.CompilerParams(
            dimension_semantics=("arbitrary",)),
    )(be, xs, gs3, W1, b1r, W2, b2r)


def _combine_body(ys_hbm, dst0_hbm, dst1_hbm, out_hbm, i0_v, i1_v, a_v, b_v, sem):
    wid = lax.axis_index("s") * NC + lax.axis_index("c")
    base = wid * TOK_W
    pltpu.sync_copy(dst0_hbm.at[pl.ds(base, TOK_W)], i0_v)
    pltpu.sync_copy(dst1_hbm.at[pl.ds(base, TOK_W)], i1_v)
    c0 = pltpu.async_copy(ys_hbm.at[i0_v], a_v, sem)
    c1 = pltpu.async_copy(ys_hbm.at[i1_v], b_v, sem)
    c0.wait()
    c1.wait()

    def body(t, carry):
        for c in range(EMBED // 16):
            sl = pl.ds(c * 16, 16)
            a_v[t, sl] = a_v[t, sl] + b_v[t, sl]
        return carry

    lax.fori_loop(0, TOK_W, body, 0)
    pltpu.sync_copy(a_v, out_hbm.at[pl.ds(base, TOK_W)])


def _combine(ys, dst0, dst1):
    mesh = plsc.VectorSubcoreMesh(core_axis_name="c", subcore_axis_name="s")
    fn = functools.partial(
        pl.kernel, mesh=mesh,
        out_type=jax.ShapeDtypeStruct((T, EMBED), jnp.float32),
        scratch_types=[
            pltpu.VMEM((TOK_W,), jnp.int32),
            pltpu.VMEM((TOK_W,), jnp.int32),
            pltpu.VMEM((TOK_W, EMBED), jnp.float32),
            pltpu.VMEM((TOK_W, EMBED), jnp.float32),
            pltpu.SemaphoreType.DMA,
        ],
    )(_combine_body)
    return fn(ys, dst0, dst1)


def _bookkeeping(gates):
    topw, topi = lax.top_k(gates, 2)
    e0, e1 = topi[:, 0], topi[:, 1]
    w0, w1 = topw[:, 0], topw[:, 1]
    oh0 = jax.nn.one_hot(e0, E, dtype=jnp.int32)
    oh1 = jax.nn.one_hot(e1, E, dtype=jnp.int32)
    oh = oh0 + oh1
    cum = jnp.cumsum(oh, axis=0) - oh
    counts = jnp.sum(oh, axis=0)
    nblk_e = (counts + BLK - 1) // BLK
    blk_start = jnp.concatenate(
        [jnp.zeros((1,), jnp.int32), jnp.cumsum(nblk_e)[:-1].astype(jnp.int32)])
    off = blk_start * BLK
    rank0 = jnp.take_along_axis(cum, e0[:, None], axis=1)[:, 0]
    rank1 = jnp.take_along_axis(cum, e1[:, None], axis=1)[:, 0]
    dst0 = (jnp.take(off, e0) + rank0).astype(jnp.int32)
    dst1 = (jnp.take(off, e1) + rank1).astype(jnp.int32)
    tok = jnp.arange(T, dtype=jnp.int32)
    src_tok = jnp.zeros((PP,), jnp.int32).at[dst0].set(tok).at[dst1].set(tok)
    g_slot = jnp.zeros((PP,), jnp.float32).at[dst0].set(w0).at[dst1].set(w1)
    bidx = jnp.arange(NB, dtype=jnp.int32)
    be = jnp.clip(jnp.sum((bidx[:, None] >= blk_start[None, :]).astype(jnp.int32),
                          axis=1) - 1, 0, E - 1).astype(jnp.int32)
    gs3 = g_slot.reshape(NB, 1, BLK)
    return src_tok, gs3, be, dst0, dst1


def kernel(x, Wg, W1, b1, W2, b2):
    orig_shape = x.shape
    xf = x.reshape(T, EMBED)
    gates = _router(xf, Wg)
    src_tok, gs3, be, dst0, dst1 = _bookkeeping(gates)
    xs = _dispatch(xf, src_tok)
    b1r = b1.reshape(E, 1, HIDDEN)
    b2r = b2.reshape(E, 1, EMBED)
    ys = _ffn(be, xs, gs3, W1, b1r, W2, b2r)
    out = _combine(ys, dst0, dst1)
    return out.reshape(orig_shape)


# speedup vs baseline: 1.1882x; 1.0255x over previous
import functools

import jax
import jax.numpy as jnp
from jax import lax
from jax.experimental import pallas as pl
from jax.experimental.pallas import tpu as pltpu
from jax.experimental.pallas import tpu_sc as plsc

EMBED = 768
HIDDEN = 3072
E = 8
T = 2048
BLK = 128
PP = T * 2 + E * BLK
NB = PP // BLK

NC, NS = 2, 16
NW = NC * NS
SLOTS_W = PP // NW
NCHUNK = 4
SCHUNK = SLOTS_W // NCHUNK
TOK_W = T // NW


def _router_body(x_ref, wg_ref, g_ref):
    logits = lax.dot_general(x_ref[...], wg_ref[...], (((1,), (1,)), ((), ())),
                             preferred_element_type=jnp.float32)
    m = jnp.max(logits, axis=1, keepdims=True)
    p = jnp.exp(logits - m)
    p = p / jnp.sum(p, axis=1, keepdims=True)
    iot = lax.broadcasted_iota(jnp.int32, (T, E), 1)
    m1 = jnp.max(p, axis=1, keepdims=True)
    i1 = jnp.min(jnp.where(p == m1, iot, E), axis=1, keepdims=True)
    sel1 = iot == i1
    p2 = jnp.where(sel1, -1.0, p)
    m2 = jnp.max(p2, axis=1, keepdims=True)
    i2 = jnp.min(jnp.where(p2 == m2, iot, E), axis=1, keepdims=True)
    sel2 = iot == i2
    den = m1 + m2 + 1e-9
    g_ref[...] = (jnp.where(sel1, m1, 0.0) + jnp.where(sel2, m2, 0.0)) / den


def _router(xf, Wg):
    return pl.pallas_call(
        _router_body,
        out_shape=jax.ShapeDtypeStruct((T, E), jnp.float32),
    )(xf, Wg)


def _dispatch_body(x_hbm, src_hbm, xs_hbm, idx_v, r0, r1, r2, r3, gsem, ssem):
    wid = lax.axis_index("s") * NC + lax.axis_index("c")
    wbase = wid * SLOTS_W
    bufs = [r0, r1, r2, r3]
    pltpu.sync_copy(src_hbm.at[pl.ds(wbase, SLOTS_W)], idx_v)
    gops = [
        pltpu.async_copy(x_hbm.at[idx_v.at[pl.ds(c * SCHUNK, SCHUNK)]],
                         bufs[c], gsem)
        for c in range(NCHUNK)
    ]
    sops = []
    for c in range(NCHUNK):
        gops[c].wait()
        sops.append(pltpu.async_copy(
            bufs[c], xs_hbm.at[pl.ds(wbase + c * SCHUNK, SCHUNK)], ssem))
    for s in sops:
        s.wait()


def _dispatch(xf, src_tok):
    mesh = plsc.VectorSubcoreMesh(core_axis_name="c", subcore_axis_name="s")
    fn = functools.partial(
        pl.kernel, mesh=mesh,
        out_type=jax.ShapeDtypeStruct((PP, EMBED), jnp.float32),
        scratch_types=[
            pltpu.VMEM((SLOTS_W,), jnp.int32),
            pltpu.VMEM((SCHUNK, EMBED), jnp.float32),
            pltpu.VMEM((SCHUNK, EMBED), jnp.float32),
            pltpu.VMEM((SCHUNK, EMBED), jnp.float32),
            pltpu.VMEM((SCHUNK, EMBED), jnp.float32),
            pltpu.SemaphoreType.DMA,
            pltpu.SemaphoreType.DMA,
        ],
    )(_dispatch_body)
    return fn(xf, src_tok)


def _ffn_body(be_ref, xs_ref, w1_ref, b1_ref, w2_ref, b2_ref, out_ref):
    h = lax.dot_general(xs_ref[...], w1_ref[0], (((1,), (1,)), ((), ())),
                        preferred_element_type=jnp.float32)
    h = jnp.maximum(h + b1_ref[0, 0][None, :], 0.0)
    y = lax.dot_general(h, w2_ref[0], (((1,), (1,)), ((), ())),
                        preferred_element_type=jnp.float32)
    out_ref[...] = y + b2_ref[0, 0][None, :]


def _ffn(be, xs, W1, b1r, W2, b2r):
    grid_spec = pltpu.PrefetchScalarGridSpec(
        num_scalar_prefetch=1,
        grid=(NB,),
        in_specs=[
            pl.BlockSpec((BLK, EMBED), lambda i, be: (i, 0)),
            pl.BlockSpec((1, HIDDEN, EMBED), lambda i, be: (be[i], 0, 0)),
            pl.BlockSpec((1, 1, HIDDEN), lambda i, be: (be[i], 0, 0)),
            pl.BlockSpec((1, EMBED, HIDDEN), lambda i, be: (be[i], 0, 0)),
            pl.BlockSpec((1, 1, EMBED), lambda i, be: (be[i], 0, 0)),
        ],
        out_specs=pl.BlockSpec((BLK, EMBED), lambda i, be: (i, 0)),
    )
    return pl.pallas_call(
        _ffn_body,
        grid_spec=grid_spec,
        out_shape=jax.ShapeDtypeStruct((PP, EMBED), jnp.float32),
        compiler_params=pltpu.CompilerParams(
            dimension_semantics=("arbitrary",)),
    )(be, xs, W1, b1r, W2, b2r)


def _combine_body(ys_hbm, dst0_hbm, dst1_hbm, w0e_hbm, w1e_hbm, out_hbm,
                  i0_v, i1_v, w0_v, w1_v, a_v, b_v, sem):
    wid = lax.axis_index("s") * NC + lax.axis_index("c")
    base = wid * TOK_W
    pltpu.sync_copy(dst0_hbm.at[pl.ds(base, TOK_W)], i0_v)
    pltpu.sync_copy(dst1_hbm.at[pl.ds(base, TOK_W)], i1_v)
    pltpu.sync_copy(w0e_hbm.at[pl.ds(base, TOK_W)], w0_v)
    pltpu.sync_copy(w1e_hbm.at[pl.ds(base, TOK_W)], w1_v)
    c0 = pltpu.async_copy(ys_hbm.at[i0_v], a_v, sem)
    c1 = pltpu.async_copy(ys_hbm.at[i1_v], b_v, sem)
    c0.wait()
    c1.wait()

    def body(t, carry):
        wv0 = w0_v[t, :]
        wv1 = w1_v[t, :]
        for c in range(EMBED // 16):
            sl = pl.ds(c * 16, 16)
            a_v[t, sl] = wv0 * a_v[t, sl] + wv1 * b_v[t, sl]
        return carry

    lax.fori_loop(0, TOK_W, body, 0)
    pltpu.sync_copy(a_v, out_hbm.at[pl.ds(base, TOK_W)])


def _combine(ys, dst0, dst1, w0e, w1e):
    mesh = plsc.VectorSubcoreMesh(core_axis_name="c", subcore_axis_name="s")
    fn = functools.partial(
        pl.kernel, mesh=mesh,
        out_type=jax.ShapeDtypeStruct((T, EMBED), jnp.float32),
        scratch_types=[
            pltpu.VMEM((TOK_W,), jnp.int32),
            pltpu.VMEM((TOK_W,), jnp.int32),
            pltpu.VMEM((TOK_W, 16), jnp.float32),
            pltpu.VMEM((TOK_W, 16), jnp.float32),
            pltpu.VMEM((TOK_W, EMBED), jnp.float32),
            pltpu.VMEM((TOK_W, EMBED), jnp.float32),
            pltpu.SemaphoreType.DMA,
        ],
    )(_combine_body)
    return fn(ys, dst0, dst1, w0e, w1e)


def _bookkeeping(gates):
    topw, topi = lax.top_k(gates, 2)
    e0, e1 = topi[:, 0], topi[:, 1]
    w0, w1 = topw[:, 0], topw[:, 1]
    oh0 = jax.nn.one_hot(e0, E, dtype=jnp.int32)
    oh1 = jax.nn.one_hot(e1, E, dtype=jnp.int32)
    oh = oh0 + oh1
    cum = jnp.cumsum(oh, axis=0) - oh
    counts = jnp.sum(oh, axis=0)
    nblk_e = (counts + BLK - 1) // BLK
    blk_start = jnp.concatenate(
        [jnp.zeros((1,), jnp.int32), jnp.cumsum(nblk_e)[:-1].astype(jnp.int32)])
    off = blk_start * BLK
    rank0 = jnp.take_along_axis(cum, e0[:, None], axis=1)[:, 0]
    rank1 = jnp.take_along_axis(cum, e1[:, None], axis=1)[:, 0]
    dst0 = (jnp.take(off, e0) + rank0).astype(jnp.int32)
    dst1 = (jnp.take(off, e1) + rank1).astype(jnp.int32)
    tok = jnp.arange(T, dtype=jnp.int32)
    src_tok = jnp.zeros((PP,), jnp.int32).at[dst0].set(tok).at[dst1].set(tok)
    bidx = jnp.arange(NB, dtype=jnp.int32)
    be = jnp.clip(jnp.sum((bidx[:, None] >= blk_start[None, :]).astype(jnp.int32),
                          axis=1) - 1, 0, E - 1).astype(jnp.int32)
    w0e = jnp.broadcast_to(w0[:, None], (T, 16))
    w1e = jnp.broadcast_to(w1[:, None], (T, 16))
    return src_tok, be, dst0, dst1, w0e, w1e


def kernel(x, Wg, W1, b1, W2, b2):
    orig_shape = x.shape
    xf = x.reshape(T, EMBED)
    gates = _router(xf, Wg)
    src_tok, be, dst0, dst1, w0e, w1e = _bookkeeping(gates)
    xs = _dispatch(xf, src_tok)
    b1r = b1.reshape(E, 1, HIDDEN)
    b2r = b2.reshape(E, 1, EMBED)
    ys = _ffn(be, xs, W1, b1r, W2, b2r)
    out = _combine(ys, dst0, dst1, w0e, w1e)
    return out.reshape(orig_shape)


# speedup vs baseline: 2.3092x; 1.9435x over previous
import functools

import jax
import jax.numpy as jnp
from jax import lax
from jax.experimental import pallas as pl
from jax.experimental.pallas import tpu as pltpu
from jax.experimental.pallas import tpu_sc as plsc

EMBED = 768
HIDDEN = 3072
E = 8
T = 2048
BLK = 256
PP = T * 2 + E * BLK
NB = PP // BLK

NC, NS = 2, 16
NW = NC * NS
TOK_W = T // NW


def _router_body(x_ref, wg_ref, d0_ref, d1_ref, w0e_ref, w1e_ref, cnt_ref):
    logits = lax.dot_general(x_ref[...], wg_ref[...], (((1,), (1,)), ((), ())),
                             preferred_element_type=jnp.float32)
    m = jnp.max(logits, axis=1, keepdims=True)
    p = jnp.exp(logits - m)
    p = p / jnp.sum(p, axis=1, keepdims=True)
    iot = lax.broadcasted_iota(jnp.int32, (T, E), 1)
    m1 = jnp.max(p, axis=1, keepdims=True)
    i1 = jnp.min(jnp.where(p == m1, iot, E), axis=1, keepdims=True)
    sel1 = (iot == i1).astype(jnp.float32)
    p2 = jnp.where(iot == i1, -1.0, p)
    m2 = jnp.max(p2, axis=1, keepdims=True)
    i2 = jnp.min(jnp.where(p2 == m2, iot, E), axis=1, keepdims=True)
    sel2 = (iot == i2).astype(jnp.float32)
    den = m1 + m2 + 1e-9
    w0e_ref[...] = jnp.broadcast_to(m1 / den, (T, 16))
    w1e_ref[...] = jnp.broadcast_to(m2 / den, (T, 16))

    oh = sel1 + sel2
    counts = jnp.sum(oh, axis=0, keepdims=True)
    nblk = jnp.floor((counts + (BLK - 1)) * (1.0 / BLK))
    eiota = lax.broadcasted_iota(jnp.int32, (E, E), 0)
    ejota = lax.broadcasted_iota(jnp.int32, (E, E), 1)
    upper = (eiota < ejota).astype(jnp.float32)
    blk_start = lax.dot_general(nblk, upper, (((1,), (0,)), ((), ())),
                                preferred_element_type=jnp.float32)
    off = blk_start * BLK
    cum = oh
    sh = 1
    while sh < T:
        cum = cum + jnp.concatenate(
            [jnp.zeros((sh, E), jnp.float32), cum[:T - sh]], axis=0)
        sh *= 2
    cum = cum - oh
    rank0 = jnp.sum(cum * sel1, axis=1, keepdims=True)
    rank1 = jnp.sum(cum * sel2, axis=1, keepdims=True)
    dst0 = jnp.sum(off * sel1, axis=1, keepdims=True) + rank0
    dst1 = jnp.sum(off * sel2, axis=1, keepdims=True) + rank1
    d0_ref[...] = dst0.astype(jnp.int32)
    d1_ref[...] = dst1.astype(jnp.int32)
    cnt_ref[...] = counts


def _router(xf, Wg):
    return pl.pallas_call(
        _router_body,
        out_shape=(
            jax.ShapeDtypeStruct((T, 1), jnp.int32),
            jax.ShapeDtypeStruct((T, 1), jnp.int32),
            jax.ShapeDtypeStruct((T, 16), jnp.float32),
            jax.ShapeDtypeStruct((T, 16), jnp.float32),
            jax.ShapeDtypeStruct((1, E), jnp.float32),
        ),
    )(xf, Wg)


def _dispatch_body(x_hbm, d0_hbm, d1_hbm, xs_hbm, i0_v, i1_v, rows_v, ssem):
    wid = lax.axis_index("s") * NC + lax.axis_index("c")
    base = wid * TOK_W
    pltpu.sync_copy(d0_hbm.at[pl.ds(base, TOK_W)], i0_v)
    pltpu.sync_copy(d1_hbm.at[pl.ds(base, TOK_W)], i1_v)
    pltpu.sync_copy(x_hbm.at[pl.ds(base, TOK_W)], rows_v)
    s0 = pltpu.async_copy(rows_v, xs_hbm.at[i0_v], ssem)
    s1 = pltpu.async_copy(rows_v, xs_hbm.at[i1_v], ssem)
    s0.wait()
    s1.wait()


def _dispatch(xf, dst0, dst1):
    mesh = plsc.VectorSubcoreMesh(core_axis_name="c", subcore_axis_name="s")
    fn = functools.partial(
        pl.kernel, mesh=mesh,
        out_type=jax.ShapeDtypeStruct((PP, EMBED), jnp.float32),
        scratch_types=[
            pltpu.VMEM((TOK_W,), jnp.int32),
            pltpu.VMEM((TOK_W,), jnp.int32),
            pltpu.VMEM((TOK_W, EMBED), jnp.float32),
            pltpu.SemaphoreType.DMA,
        ],
    )(_dispatch_body)
    return fn(xf, dst0, dst1)


def _ffn_body(be_ref, xs_ref, w1_ref, b1_ref, w2_ref, b2_ref, out_ref):
    h = lax.dot_general(xs_ref[...], w1_ref[0], (((1,), (1,)), ((), ())),
                        preferred_element_type=jnp.float32)
    h = jnp.maximum(h + b1_ref[0, 0][None, :], 0.0)
    y = lax.dot_general(h, w2_ref[0], (((1,), (1,)), ((), ())),
                        preferred_element_type=jnp.float32)
    out_ref[...] = y + b2_ref[0, 0][None, :]


def _ffn(be, xs, W1, b1r, W2, b2r):
    grid_spec = pltpu.PrefetchScalarGridSpec(
        num_scalar_prefetch=1,
        grid=(NB,),
        in_specs=[
            pl.BlockSpec((BLK, EMBED), lambda i, be: (i, 0)),
            pl.BlockSpec((1, HIDDEN, EMBED), lambda i, be: (be[i], 0, 0)),
            pl.BlockSpec((1, 1, HIDDEN), lambda i, be: (be[i], 0, 0)),
            pl.BlockSpec((1, EMBED, HIDDEN), lambda i, be: (be[i], 0, 0)),
            pl.BlockSpec((1, 1, EMBED), lambda i, be: (be[i], 0, 0)),
        ],
        out_specs=pl.BlockSpec((BLK, EMBED), lambda i, be: (i, 0)),
    )
    return pl.pallas_call(
        _ffn_body,
        grid_spec=grid_spec,
        out_shape=jax.ShapeDtypeStruct((PP, EMBED), jnp.float32),
        compiler_params=pltpu.CompilerParams(
            dimension_semantics=("arbitrary",)),
    )(be, xs, W1, b1r, W2, b2r)


def _combine_body(ys_hbm, dst0_hbm, dst1_hbm, w0e_hbm, w1e_hbm, out_hbm,
                  i0_v, i1_v, w0_v, w1_v, a_v, b_v, sem):
    wid = lax.axis_index("s") * NC + lax.axis_index("c")
    base = wid * TOK_W
    pltpu.sync_copy(dst0_hbm.at[pl.ds(base, TOK_W)], i0_v)
    pltpu.sync_copy(dst1_hbm.at[pl.ds(base, TOK_W)], i1_v)
    pltpu.sync_copy(w0e_hbm.at[pl.ds(base, TOK_W)], w0_v)
    pltpu.sync_copy(w1e_hbm.at[pl.ds(base, TOK_W)], w1_v)
    c0 = pltpu.async_copy(ys_hbm.at[i0_v], a_v, sem)
    c1 = pltpu.async_copy(ys_hbm.at[i1_v], b_v, sem)
    c0.wait()
    c1.wait()

    def body(t, carry):
        wv0 = w0_v[t, :]
        wv1 = w1_v[t, :]
        for c in range(EMBED // 16):
            sl = pl.ds(c * 16, 16)
            a_v[t, sl] = wv0 * a_v[t, sl] + wv1 * b_v[t, sl]
        return carry

    lax.fori_loop(0, TOK_W, body, 0)
    pltpu.sync_copy(a_v, out_hbm.at[pl.ds(base, TOK_W)])


def _combine(ys, dst0, dst1, w0e, w1e):
    mesh = plsc.VectorSubcoreMesh(core_axis_name="c", subcore_axis_name="s")
    fn = functools.partial(
        pl.kernel, mesh=mesh,
        out_type=jax.ShapeDtypeStruct((T, EMBED), jnp.float32),
        scratch_types=[
            pltpu.VMEM((TOK_W,), jnp.int32),
            pltpu.VMEM((TOK_W,), jnp.int32),
            pltpu.VMEM((TOK_W, 16), jnp.float32),
            pltpu.VMEM((TOK_W, 16), jnp.float32),
            pltpu.VMEM((TOK_W, EMBED), jnp.float32),
            pltpu.VMEM((TOK_W, EMBED), jnp.float32),
            pltpu.SemaphoreType.DMA,
        ],
    )(_combine_body)
    return fn(ys, dst0, dst1, w0e, w1e)


def kernel(x, Wg, W1, b1, W2, b2):
    orig_shape = x.shape
    xf = x.reshape(T, EMBED)
    d0c, d1c, w0e, w1e, counts = _router(xf, Wg)
    dst0 = d0c.reshape(T)
    dst1 = d1c.reshape(T)
    nblk = jnp.ceil(counts[0] * (1.0 / BLK)).astype(jnp.int32)
    blk_start = jnp.concatenate(
        [jnp.zeros((1,), jnp.int32), jnp.cumsum(nblk)[:-1].astype(jnp.int32)])
    bidx = jnp.arange(NB, dtype=jnp.int32)
    be = jnp.clip(jnp.sum((bidx[:, None] >= blk_start[None, :]).astype(jnp.int32),
                          axis=1) - 1, 0, E - 1).astype(jnp.int32)
    xs = _dispatch(xf, dst0, dst1)
    b1r = b1.reshape(E, 1, HIDDEN)
    b2r = b2.reshape(E, 1, EMBED)
    ys = _ffn(be, xs, W1, b1r, W2, b2r)
    out = _combine(ys, dst0, dst1, w0e, w1e)
    return out.reshape(orig_shape)


# speedup vs baseline: 2.4021x; 1.0403x over previous
import functools

import jax
import jax.numpy as jnp
from jax import lax
from jax.experimental import pallas as pl
from jax.experimental.pallas import tpu as pltpu
from jax.experimental.pallas import tpu_sc as plsc

EMBED = 768
HIDDEN = 3072
E = 8
T = 2048
BLK = 256
PP = T * 2 + E * BLK
NB = PP // BLK

NC, NS = 2, 16
NW = NC * NS
TOK_W = T // NW


def _router_body(x_ref, wg_ref, d0_ref, d1_ref, w0e_ref, w1e_ref, cnt_ref):
    logits = lax.dot_general(x_ref[...], wg_ref[...], (((1,), (1,)), ((), ())),
                             preferred_element_type=jnp.float32)
    m = jnp.max(logits, axis=1, keepdims=True)
    p = jnp.exp(logits - m)
    p = p / jnp.sum(p, axis=1, keepdims=True)
    iot = lax.broadcasted_iota(jnp.int32, (T, E), 1)
    m1 = jnp.max(p, axis=1, keepdims=True)
    i1 = jnp.min(jnp.where(p == m1, iot, E), axis=1, keepdims=True)
    sel1 = (iot == i1).astype(jnp.float32)
    p2 = jnp.where(iot == i1, -1.0, p)
    m2 = jnp.max(p2, axis=1, keepdims=True)
    i2 = jnp.min(jnp.where(p2 == m2, iot, E), axis=1, keepdims=True)
    sel2 = (iot == i2).astype(jnp.float32)
    den = m1 + m2 + 1e-9
    w0e_ref[...] = jnp.broadcast_to(m1 / den, (T, 16))
    w1e_ref[...] = jnp.broadcast_to(m2 / den, (T, 16))

    oh = sel1 + sel2
    counts = jnp.sum(oh, axis=0, keepdims=True)
    nblk = jnp.floor((counts + (BLK - 1)) * (1.0 / BLK))
    eiota = lax.broadcasted_iota(jnp.int32, (E, E), 0)
    ejota = lax.broadcasted_iota(jnp.int32, (E, E), 1)
    upper = (eiota < ejota).astype(jnp.float32)
    blk_start = lax.dot_general(nblk, upper, (((1,), (0,)), ((), ())),
                                preferred_element_type=jnp.float32)
    off = blk_start * BLK
    cum = oh
    sh = 1
    while sh < T:
        cum = cum + jnp.concatenate(
            [jnp.zeros((sh, E), jnp.float32), cum[:T - sh]], axis=0)
        sh *= 2
    cum = cum - oh
    rank0 = jnp.sum(cum * sel1, axis=1, keepdims=True)
    rank1 = jnp.sum(cum * sel2, axis=1, keepdims=True)
    dst0 = jnp.sum(off * sel1, axis=1, keepdims=True) + rank0
    dst1 = jnp.sum(off * sel2, axis=1, keepdims=True) + rank1
    d0_ref[...] = dst0.astype(jnp.int32)
    d1_ref[...] = dst1.astype(jnp.int32)
    cnt_ref[...] = counts


def _router(xf, Wg):
    return pl.pallas_call(
        _router_body,
        out_shape=(
            jax.ShapeDtypeStruct((T, 1), jnp.int32),
            jax.ShapeDtypeStruct((T, 1), jnp.int32),
            jax.ShapeDtypeStruct((T, 16), jnp.float32),
            jax.ShapeDtypeStruct((T, 16), jnp.float32),
            jax.ShapeDtypeStruct((1, E), jnp.float32),
        ),
    )(xf, Wg)


def _dispatch_body(x_hbm, d0_hbm, d1_hbm, xs_hbm, i0_v, i1_v, rows_v, ssem):
    wid = lax.axis_index("s") * NC + lax.axis_index("c")
    base = wid * TOK_W
    pltpu.sync_copy(d0_hbm.at[pl.ds(base, TOK_W)], i0_v)
    pltpu.sync_copy(d1_hbm.at[pl.ds(base, TOK_W)], i1_v)
    pltpu.sync_copy(x_hbm.at[pl.ds(base, TOK_W)], rows_v)
    s0 = pltpu.async_copy(rows_v, xs_hbm.at[i0_v], ssem)
    s1 = pltpu.async_copy(rows_v, xs_hbm.at[i1_v], ssem)
    s0.wait()
    s1.wait()


def _dispatch(xf, dst0, dst1):
    mesh = plsc.VectorSubcoreMesh(core_axis_name="c", subcore_axis_name="s")
    fn = functools.partial(
        pl.kernel, mesh=mesh,
        out_type=jax.ShapeDtypeStruct((PP, EMBED), jnp.float32),
        scratch_types=[
            pltpu.VMEM((TOK_W,), jnp.int32),
            pltpu.VMEM((TOK_W,), jnp.int32),
            pltpu.VMEM((TOK_W, EMBED), jnp.float32),
            pltpu.SemaphoreType.DMA,
        ],
    )(_dispatch_body)
    return fn(xf, dst0, dst1)


def _ffn_body(be_ref, xs_ref, w1_ref, b1_ref, w2_ref, b2_ref, out_ref):
    i = pl.program_id(0)

    @pl.when(i < be_ref[NB])
    def _():
        h = lax.dot_general(xs_ref[...], w1_ref[0], (((1,), (1,)), ((), ())),
                            preferred_element_type=jnp.float32)
        h = jnp.maximum(h + b1_ref[0, 0][None, :], 0.0)
        y = lax.dot_general(h, w2_ref[0], (((1,), (1,)), ((), ())),
                            preferred_element_type=jnp.float32)
        out_ref[...] = y + b2_ref[0, 0][None, :]


def _ffn(be, xs, W1, b1r, W2, b2r):
    grid_spec = pltpu.PrefetchScalarGridSpec(
        num_scalar_prefetch=1,
        grid=(NB,),
        in_specs=[
            pl.BlockSpec((BLK, EMBED), lambda i, be: (i, 0)),
            pl.BlockSpec((1, HIDDEN, EMBED), lambda i, be: (be[i], 0, 0)),
            pl.BlockSpec((1, 1, HIDDEN), lambda i, be: (be[i], 0, 0)),
            pl.BlockSpec((1, EMBED, HIDDEN), lambda i, be: (be[i], 0, 0)),
            pl.BlockSpec((1, 1, EMBED), lambda i, be: (be[i], 0, 0)),
        ],
        out_specs=pl.BlockSpec((BLK, EMBED), lambda i, be: (i, 0)),
    )
    return pl.pallas_call(
        _ffn_body,
        grid_spec=grid_spec,
        out_shape=jax.ShapeDtypeStruct((PP, EMBED), jnp.float32),
        compiler_params=pltpu.CompilerParams(
            dimension_semantics=("arbitrary",)),
    )(be, xs, W1, b1r, W2, b2r)


def _combine_body(ys_hbm, dst0_hbm, dst1_hbm, w0e_hbm, w1e_hbm, out_hbm,
                  i0_v, i1_v, w0_v, w1_v, a_v, b_v, sem):
    wid = lax.axis_index("s") * NC + lax.axis_index("c")
    base = wid * TOK_W
    pltpu.sync_copy(dst0_hbm.at[pl.ds(base, TOK_W)], i0_v)
    pltpu.sync_copy(dst1_hbm.at[pl.ds(base, TOK_W)], i1_v)
    pltpu.sync_copy(w0e_hbm.at[pl.ds(base, TOK_W)], w0_v)
    pltpu.sync_copy(w1e_hbm.at[pl.ds(base, TOK_W)], w1_v)
    c0 = pltpu.async_copy(ys_hbm.at[i0_v], a_v, sem)
    c1 = pltpu.async_copy(ys_hbm.at[i1_v], b_v, sem)
    c0.wait()
    c1.wait()

    def body(t, carry):
        wv0 = w0_v[t, :]
        wv1 = w1_v[t, :]
        for c in range(EMBED // 16):
            sl = pl.ds(c * 16, 16)
            a_v[t, sl] = wv0 * a_v[t, sl] + wv1 * b_v[t, sl]
        return carry

    lax.fori_loop(0, TOK_W, body, 0)
    pltpu.sync_copy(a_v, out_hbm.at[pl.ds(base, TOK_W)])


def _combine(ys, dst0, dst1, w0e, w1e):
    mesh = plsc.VectorSubcoreMesh(core_axis_name="c", subcore_axis_name="s")
    fn = functools.partial(
        pl.kernel, mesh=mesh,
        out_type=jax.ShapeDtypeStruct((T, EMBED), jnp.float32),
        scratch_types=[
            pltpu.VMEM((TOK_W,), jnp.int32),
            pltpu.VMEM((TOK_W,), jnp.int32),
            pltpu.VMEM((TOK_W, 16), jnp.float32),
            pltpu.VMEM((TOK_W, 16), jnp.float32),
            pltpu.VMEM((TOK_W, EMBED), jnp.float32),
            pltpu.VMEM((TOK_W, EMBED), jnp.float32),
            pltpu.SemaphoreType.DMA,
        ],
    )(_combine_body)
    return fn(ys, dst0, dst1, w0e, w1e)


def kernel(x, Wg, W1, b1, W2, b2):
    orig_shape = x.shape
    xf = x.reshape(T, EMBED)
    d0c, d1c, w0e, w1e, counts = _router(xf, Wg)
    dst0 = d0c.reshape(T)
    dst1 = d1c.reshape(T)
    nblk = jnp.ceil(counts[0] * (1.0 / BLK)).astype(jnp.int32)
    blk_start = jnp.concatenate(
        [jnp.zeros((1,), jnp.int32), jnp.cumsum(nblk)[:-1].astype(jnp.int32)])
    bidx = jnp.arange(NB, dtype=jnp.int32)
    be = jnp.clip(jnp.sum((bidx[:, None] >= blk_start[None, :]).astype(jnp.int32),
                          axis=1) - 1, 0, E - 1).astype(jnp.int32)
    be = jnp.concatenate([be, jnp.sum(nblk, keepdims=True)])
    xs = _dispatch(xf, dst0, dst1)
    b1r = b1.reshape(E, 1, HIDDEN)
    b2r = b2.reshape(E, 1, EMBED)
    ys = _ffn(be, xs, W1, b1r, W2, b2r)
    out = _combine(ys, dst0, dst1, w0e, w1e)
    return out.reshape(orig_shape)


# speedup vs baseline: 2.5938x; 1.0798x over previous
import functools

import jax
import jax.numpy as jnp
from jax import lax
from jax.experimental import pallas as pl
from jax.experimental.pallas import tpu as pltpu
from jax.experimental.pallas import tpu_sc as plsc

EMBED = 768
HIDDEN = 3072
E = 8
T = 2048
BLK = 512
PP = T * 2 + E * BLK
NB = PP // BLK

NC, NS = 2, 16
NW = NC * NS
TOK_W = T // NW


def _router_body(x_ref, wg_ref, d0_ref, d1_ref, w0e_ref, w1e_ref, cnt_ref):
    logits = lax.dot_general(x_ref[...], wg_ref[...], (((1,), (1,)), ((), ())),
                             preferred_element_type=jnp.float32)
    m = jnp.max(logits, axis=1, keepdims=True)
    p = jnp.exp(logits - m)
    p = p / jnp.sum(p, axis=1, keepdims=True)
    iot = lax.broadcasted_iota(jnp.int32, (T, E), 1)
    m1 = jnp.max(p, axis=1, keepdims=True)
    i1 = jnp.min(jnp.where(p == m1, iot, E), axis=1, keepdims=True)
    sel1 = (iot == i1).astype(jnp.float32)
    p2 = jnp.where(iot == i1, -1.0, p)
    m2 = jnp.max(p2, axis=1, keepdims=True)
    i2 = jnp.min(jnp.where(p2 == m2, iot, E), axis=1, keepdims=True)
    sel2 = (iot == i2).astype(jnp.float32)
    den = m1 + m2 + 1e-9
    w0e_ref[...] = jnp.broadcast_to(m1 / den, (T, 16))
    w1e_ref[...] = jnp.broadcast_to(m2 / den, (T, 16))

    oh = sel1 + sel2
    counts = jnp.sum(oh, axis=0, keepdims=True)
    nblk = jnp.floor((counts + (BLK - 1)) * (1.0 / BLK))
    eiota = lax.broadcasted_iota(jnp.int32, (E, E), 0)
    ejota = lax.broadcasted_iota(jnp.int32, (E, E), 1)
    upper = (eiota < ejota).astype(jnp.float32)
    blk_start = lax.dot_general(nblk, upper, (((1,), (0,)), ((), ())),
                                preferred_element_type=jnp.float32)
    off = blk_start * BLK
    cum = oh
    sh = 1
    while sh < T:
        cum = cum + jnp.concatenate(
            [jnp.zeros((sh, E), jnp.float32), cum[:T - sh]], axis=0)
        sh *= 2
    cum = cum - oh
    rank0 = jnp.sum(cum * sel1, axis=1, keepdims=True)
    rank1 = jnp.sum(cum * sel2, axis=1, keepdims=True)
    dst0 = jnp.sum(off * sel1, axis=1, keepdims=True) + rank0
    dst1 = jnp.sum(off * sel2, axis=1, keepdims=True) + rank1
    d0_ref[...] = dst0.astype(jnp.int32)
    d1_ref[...] = dst1.astype(jnp.int32)
    cnt_ref[...] = counts


def _router(xf, Wg):
    return pl.pallas_call(
        _router_body,
        out_shape=(
            jax.ShapeDtypeStruct((T, 1), jnp.int32),
            jax.ShapeDtypeStruct((T, 1), jnp.int32),
            jax.ShapeDtypeStruct((T, 16), jnp.float32),
            jax.ShapeDtypeStruct((T, 16), jnp.float32),
            jax.ShapeDtypeStruct((1, E), jnp.float32),
        ),
    )(xf, Wg)


def _dispatch_body(x_hbm, d0_hbm, d1_hbm, xs_hbm, i0_v, i1_v, rows_v, ssem):
    wid = lax.axis_index("s") * NC + lax.axis_index("c")
    base = wid * TOK_W
    pltpu.sync_copy(d0_hbm.at[pl.ds(base, TOK_W)], i0_v)
    pltpu.sync_copy(d1_hbm.at[pl.ds(base, TOK_W)], i1_v)
    pltpu.sync_copy(x_hbm.at[pl.ds(base, TOK_W)], rows_v)
    s0 = pltpu.async_copy(rows_v, xs_hbm.at[i0_v], ssem)
    s1 = pltpu.async_copy(rows_v, xs_hbm.at[i1_v], ssem)
    s0.wait()
    s1.wait()


def _dispatch(xf, dst0, dst1):
    mesh = plsc.VectorSubcoreMesh(core_axis_name="c", subcore_axis_name="s")
    fn = functools.partial(
        pl.kernel, mesh=mesh,
        out_type=jax.ShapeDtypeStruct((PP, EMBED), jnp.float32),
        scratch_types=[
            pltpu.VMEM((TOK_W,), jnp.int32),
            pltpu.VMEM((TOK_W,), jnp.int32),
            pltpu.VMEM((TOK_W, EMBED), jnp.float32),
            pltpu.SemaphoreType.DMA,
        ],
    )(_dispatch_body)
    return fn(xf, dst0, dst1)


def _ffn_body(be_ref, xs_ref, w1_ref, b1_ref, w2_ref, b2_ref, out_ref):
    i = pl.program_id(0)

    @pl.when(i < be_ref[NB])
    def _():
        h = lax.dot_general(xs_ref[...], w1_ref[0], (((1,), (1,)), ((), ())),
                            preferred_element_type=jnp.float32)
        h = jnp.maximum(h + b1_ref[0, 0][None, :], 0.0)
        y = lax.dot_general(h, w2_ref[0], (((1,), (1,)), ((), ())),
                            preferred_element_type=jnp.float32)
        out_ref[...] = y + b2_ref[0, 0][None, :]


def _ffn(be, xs, W1, b1r, W2, b2r):
    grid_spec = pltpu.PrefetchScalarGridSpec(
        num_scalar_prefetch=1,
        grid=(NB,),
        in_specs=[
            pl.BlockSpec((BLK, EMBED), lambda i, be: (i, 0)),
            pl.BlockSpec((1, HIDDEN, EMBED), lambda i, be: (be[i], 0, 0)),
            pl.BlockSpec((1, 1, HIDDEN), lambda i, be: (be[i], 0, 0)),
            pl.BlockSpec((1, EMBED, HIDDEN), lambda i, be: (be[i], 0, 0)),
            pl.BlockSpec((1, 1, EMBED), lambda i, be: (be[i], 0, 0)),
        ],
        out_specs=pl.BlockSpec((BLK, EMBED), lambda i, be: (i, 0)),
    )
    return pl.pallas_call(
        _ffn_body,
        grid_spec=grid_spec,
        out_shape=jax.ShapeDtypeStruct((PP, EMBED), jnp.float32),
        compiler_params=pltpu.CompilerParams(
            dimension_semantics=("arbitrary",)),
    )(be, xs, W1, b1r, W2, b2r)


def _combine_body(ys_hbm, dst0_hbm, dst1_hbm, w0e_hbm, w1e_hbm, out_hbm,
                  i0_v, i1_v, w0_v, w1_v, a_v, b_v, sem):
    wid = lax.axis_index("s") * NC + lax.axis_index("c")
    base = wid * TOK_W
    pltpu.sync_copy(dst0_hbm.at[pl.ds(base, TOK_W)], i0_v)
    pltpu.sync_copy(dst1_hbm.at[pl.ds(base, TOK_W)], i1_v)
    pltpu.sync_copy(w0e_hbm.at[pl.ds(base, TOK_W)], w0_v)
    pltpu.sync_copy(w1e_hbm.at[pl.ds(base, TOK_W)], w1_v)
    c0 = pltpu.async_copy(ys_hbm.at[i0_v], a_v, sem)
    c1 = pltpu.async_copy(ys_hbm.at[i1_v], b_v, sem)
    c0.wait()
    c1.wait()

    def body(t, carry):
        wv0 = w0_v[t, :]
        wv1 = w1_v[t, :]
        for c in range(EMBED // 16):
            sl = pl.ds(c * 16, 16)
            a_v[t, sl] = wv0 * a_v[t, sl] + wv1 * b_v[t, sl]
        return carry

    lax.fori_loop(0, TOK_W, body, 0)
    pltpu.sync_copy(a_v, out_hbm.at[pl.ds(base, TOK_W)])


def _combine(ys, dst0, dst1, w0e, w1e):
    mesh = plsc.VectorSubcoreMesh(core_axis_name="c", subcore_axis_name="s")
    fn = functools.partial(
        pl.kernel, mesh=mesh,
        out_type=jax.ShapeDtypeStruct((T, EMBED), jnp.float32),
        scratch_types=[
            pltpu.VMEM((TOK_W,), jnp.int32),
            pltpu.VMEM((TOK_W,), jnp.int32),
            pltpu.VMEM((TOK_W, 16), jnp.float32),
            pltpu.VMEM((TOK_W, 16), jnp.float32),
            pltpu.VMEM((TOK_W, EMBED), jnp.float32),
            pltpu.VMEM((TOK_W, EMBED), jnp.float32),
            pltpu.SemaphoreType.DMA,
        ],
    )(_combine_body)
    return fn(ys, dst0, dst1, w0e, w1e)


def kernel(x, Wg, W1, b1, W2, b2):
    orig_shape = x.shape
    xf = x.reshape(T, EMBED)
    d0c, d1c, w0e, w1e, counts = _router(xf, Wg)
    dst0 = d0c.reshape(T)
    dst1 = d1c.reshape(T)
    nblk = jnp.ceil(counts[0] * (1.0 / BLK)).astype(jnp.int32)
    blk_start = jnp.concatenate(
        [jnp.zeros((1,), jnp.int32), jnp.cumsum(nblk)[:-1].astype(jnp.int32)])
    bidx = jnp.arange(NB, dtype=jnp.int32)
    be = jnp.clip(jnp.sum((bidx[:, None] >= blk_start[None, :]).astype(jnp.int32),
                          axis=1) - 1, 0, E - 1).astype(jnp.int32)
    be = jnp.concatenate([be, jnp.sum(nblk, keepdims=True)])
    xs = _dispatch(xf, dst0, dst1)
    b1r = b1.reshape(E, 1, HIDDEN)
    b2r = b2.reshape(E, 1, EMBED)
    ys = _ffn(be, xs, W1, b1r, W2, b2r)
    out = _combine(ys, dst0, dst1, w0e, w1e)
    return out.reshape(orig_shape)
